# Initial kernel scaffold; baseline (speedup 1.0000x reference)
import jax
import jax.numpy as jnp
from jax.experimental import pallas as pl


def kernel(h, edge_index, edge_lbl, W):
    raise NotImplementedError("write your pallas kernel here")


# speedup vs baseline: 2.4184x; 2.4184x over previous
import functools

import jax
import jax.numpy as jnp
from jax import lax
from jax.experimental import pallas as pl
from jax.experimental.pallas import tpu as pltpu
from jax.experimental.pallas import tpu_sc as plsc

N_NODES = 10000
NODE_DIM = 128
HALF = NODE_DIM // 2
LBL_DIM = 16
Z_DIM = 128
N_EDGES = 320000

NC = 2
NS = 16
ROW = 128
EP = 323584
NROWS = EP // ROW
RPT = NROWS // (NC * NS)
RPT1 = NROWS // NS
NACC = 10240
NPT = NACC // NS
NTAB = N_NODES + 1
NB = 327680
BPT = NB // NS
SENT = NB - 1


def _fill2d(ref, rows, cols, val, dtype):
  v = jnp.full((16,), val, dtype)

  def body(i, _):
    for j in range(cols // 16):
      ref[i, pl.ds(j * 16, 16)] = v
    return 0

  lax.fori_loop(0, rows, body, 0)


def _fill1d(ref, n, val, dtype):
  v = jnp.full((16,), val, dtype)

  def body(i, _):
    ref[pl.ds(i * 16, 16)] = v
    return 0

  lax.fori_loop(0, n // 16, body, 0)


def _agg_body(h2_hbm, srci_hbm, dsti_hbm, lbl_hbm,
              ph_hbm, plb_hbm,
              src_v, dst_v, hbuf, lbuf, sem,
              agg_h, agg_l):
  c = lax.axis_index("c")
  s = lax.axis_index("s")
  w = c * NS + s
  off = c * NTAB

  _fill2d(hbuf, ROW, HALF, 0.0, jnp.float32)
  _fill2d(lbuf, ROW, LBL_DIM, 0.0, jnp.float32)

  nb = s * NPT
  for k in range(NPT // ROW):
    sl = pl.ds(nb + k * ROW, ROW)
    pltpu.sync_copy(hbuf, agg_h.at[sl])
    pltpu.sync_copy(lbuf, agg_l.at[sl])
  plsc.subcore_barrier()

  def h_step(j, _):
    r = s * RPT1 + j
    pltpu.sync_copy(srci_hbm.at[r], src_v)
    pltpu.sync_copy(dsti_hbm.at[r], dst_v)
    for jj in range(ROW // 16):
      sl = pl.ds(jj * 16, 16)
      src_v[sl] = src_v[sl] + off
    pltpu.async_copy(h2_hbm.at[src_v], hbuf, sem).wait()
    pltpu.sync_copy(hbuf, agg_h.at[dst_v], add=True)
    return 0

  lax.fori_loop(0, RPT1, h_step, 0)

  def l_step(j, _):
    r = w * RPT + j
    pltpu.sync_copy(dsti_hbm.at[r], dst_v)
    pltpu.sync_copy(lbl_hbm.at[pl.ds(r * ROW, ROW)], lbuf)
    pltpu.sync_copy(lbuf, agg_l.at[dst_v], add=True)
    return 0

  lax.fori_loop(0, RPT, l_step, 0)
  plsc.subcore_barrier()

  for k in range(NPT // ROW):
    sl = pl.ds(nb + k * ROW, ROW)
    pltpu.sync_copy(agg_h.at[sl], ph_hbm.at[c, sl])
    pltpu.sync_copy(agg_l.at[sl], plb_hbm.at[c, sl])


_agg = pl.kernel(
    _agg_body,
    out_type=[
        jax.ShapeDtypeStruct((NC, NACC, HALF), jnp.float32),
        jax.ShapeDtypeStruct((NC, NACC, LBL_DIM), jnp.float32),
    ],
    mesh=plsc.VectorSubcoreMesh(core_axis_name="c", subcore_axis_name="s"),
    compiler_params=pltpu.CompilerParams(use_tc_tiling_on_sc=False),
    scratch_types=[
        pltpu.VMEM((ROW,), jnp.int32),
        pltpu.VMEM((ROW,), jnp.int32),
        pltpu.VMEM((ROW, HALF), jnp.float32),
        pltpu.VMEM((ROW, LBL_DIM), jnp.float32),
        pltpu.SemaphoreType.DMA,
        pltpu.VMEM_SHARED((NACC, HALF), jnp.float32),
        pltpu.VMEM_SHARED((NACC, LBL_DIM), jnp.float32),
    ],
)


def _denom_body(dsti_hbm, inv_hbm, bkt_hbm,
                dst_v, ones_v, zb, dvbuf, degi, dbuf, invbuf, sem,
                bucket_sp, deg_sp):
  s = lax.axis_index("s")

  _fill1d(zb, 2048, 0.0, jnp.float32)
  _fill1d(ones_v, ROW, 1.0, jnp.float32)
  for k in range(BPT // 2048):
    pltpu.sync_copy(zb, bucket_sp.at[pl.ds(s * BPT + k * 2048, 2048)])
  pltpu.sync_copy(zb.at[pl.ds(0, NPT)], deg_sp.at[pl.ds(s * NPT, NPT)])
  plsc.subcore_barrier()

  def edge_step(j, _):
    r = s * RPT1 + j
    pltpu.sync_copy(dsti_hbm.at[r], dst_v)
    pltpu.sync_copy(ones_v, deg_sp.at[dst_v], add=True)
    return 0

  lax.fori_loop(0, RPT1, edge_step, 0)
  plsc.subcore_barrier()

  n0 = s * NPT
  pltpu.sync_copy(deg_sp.at[pl.ds(n0, NPT)], dvbuf)
  for i in range(NPT // 16):
    x = dvbuf[pl.ds(i * 16, 16)]
    n = n0 + i * 16 + lax.iota(jnp.int32, 16)
    idx = jnp.where(n < N_NODES, x.astype(jnp.int32), SENT)
    degi[i // 8, pl.ds((i % 8) * 16, 16)] = idx

  for k in range(NPT // ROW):
    pltpu.sync_copy(ones_v, bucket_sp.at[degi.at[k]], add=True)
  plsc.subcore_barrier()

  pltpu.sync_copy(bucket_sp.at[pl.ds(s * BPT, BPT)], bkt_hbm.at[pl.ds(s * BPT, BPT)])
  plsc.subcore_barrier()

  for k in range(NPT // ROW):
    pltpu.async_copy(bkt_hbm.at[degi.at[k]], dbuf, sem).wait()
    for i in range(ROW // 16):
      d = dbuf[pl.ds(i * 16, 16)]
      invbuf[pl.ds((k * 8 + i) * 16, 16)] = 1.0 / d
  pltpu.sync_copy(invbuf, inv_hbm.at[pl.ds(n0, NPT)])


_denom = pl.kernel(
    _denom_body,
    out_type=[
        jax.ShapeDtypeStruct((NACC,), jnp.float32),
        jax.ShapeDtypeStruct((NB,), jnp.float32),
    ],
    mesh=plsc.VectorSubcoreMesh(core_axis_name="c", subcore_axis_name="s"),
    scratch_types=[
        pltpu.VMEM((ROW,), jnp.int32),
        pltpu.VMEM((ROW,), jnp.float32),
        pltpu.VMEM((2048,), jnp.float32),
        pltpu.VMEM((NPT,), jnp.float32),
        pltpu.VMEM((NPT // ROW, ROW), jnp.int32),
        pltpu.VMEM((ROW,), jnp.float32),
        pltpu.VMEM((NPT,), jnp.float32),
        pltpu.SemaphoreType.DMA,
        pltpu.VMEM_SHARED((NB,), jnp.float32),
        pltpu.VMEM_SHARED((NACC,), jnp.float32),
    ],
)


def _linear_body(ph, plb, inv, w1, w2, o):
  b = plb[0] + plb[1]
  acc = lax.dot_general(ph[0], w1[:, :HALF], (((1,), (1,)), ((), ())),
                        preferred_element_type=jnp.float32,
                        precision=lax.Precision.HIGHEST)
  acc = acc + lax.dot_general(ph[1], w1[:, HALF:], (((1,), (1,)), ((), ())),
                              preferred_element_type=jnp.float32,
                              precision=lax.Precision.HIGHEST)
  acc = acc + lax.dot_general(b, w2[...], (((1,), (1,)), ((), ())),
                              preferred_element_type=jnp.float32,
                              precision=lax.Precision.HIGHEST)
  acc = acc * inv[...]
  o[...] = jnp.where(acc > 0, acc, jnp.exp(jnp.minimum(acc, 0.0)) - 1.0)


_TC_BLK = 1024


def _linear(ph, plb, inv, w1, w2):
  grid = (NACC // _TC_BLK,)
  return pl.pallas_call(
      _linear_body,
      grid=grid,
      in_specs=[
          pl.BlockSpec((NC, _TC_BLK, HALF), lambda i: (0, i, 0)),
          pl.BlockSpec((NC, _TC_BLK, LBL_DIM), lambda i: (0, i, 0)),
          pl.BlockSpec((_TC_BLK, 1), lambda i: (i, 0)),
          pl.BlockSpec((Z_DIM, NODE_DIM), lambda i: (0, 0)),
          pl.BlockSpec((Z_DIM, LBL_DIM), lambda i: (0, 0)),
      ],
      out_specs=pl.BlockSpec((_TC_BLK, Z_DIM), lambda i: (i, 0)),
      out_shape=jax.ShapeDtypeStruct((NACC, Z_DIM), jnp.float32),
  )(ph, plb, inv, w1, w2)


@jax.jit
def kernel(h, edge_index, edge_lbl, W):
  pad = EP - N_EDGES
  src = edge_index[0]
  dst = edge_index[1]
  fill = jnp.full((pad,), N_NODES, jnp.int32)
  srcp = jnp.concatenate([src, fill]).reshape(NROWS, ROW)
  dstp = jnp.concatenate([dst, fill]).reshape(NROWS, ROW)
  lblp = jnp.concatenate(
      [edge_lbl, jnp.zeros((pad, LBL_DIM), jnp.float32)], axis=0)
  hp = jnp.concatenate([h, jnp.zeros((1, NODE_DIM), jnp.float32)], axis=0)
  h2 = jnp.concatenate([hp[:, :HALF], hp[:, HALF:]], axis=0)

  ph, plb = _agg(h2, srcp, dstp, lblp)
  inv, _ = _denom(dstp)

  z = _linear(ph, plb, inv.reshape(NACC, 1), W[:, :NODE_DIM], W[:, NODE_DIM:])
  return z[:N_NODES]


# speedup vs baseline: 2.9751x; 1.2302x over previous
import functools

import jax
import jax.numpy as jnp
from jax import lax
from jax.experimental import pallas as pl
from jax.experimental.pallas import tpu as pltpu
from jax.experimental.pallas import tpu_sc as plsc

N_NODES = 10000
NODE_DIM = 128
HALF = NODE_DIM // 2
LBL_DIM = 16
Z_DIM = 128
N_EDGES = 320000

NC = 2
NS = 16
ROW = 128
EP = 327680
NROWS = EP // ROW
RPT = NROWS // (NC * NS)
RPT1 = NROWS // NS
G = 4
LG = 2
NACC = 10240
NPT = NACC // NS
NTAB = N_NODES + 1
NB = 327680
BPT = NB // NS
SENT = NB - 1


def _fill2d(ref, rows, cols, val, dtype):
  v = jnp.full((16,), val, dtype)

  def body(i, _):
    for j in range(cols // 16):
      ref[i, pl.ds(j * 16, 16)] = v
    return 0

  lax.fori_loop(0, rows, body, 0)


def _fill1d(ref, n, val, dtype):
  v = jnp.full((16,), val, dtype)

  def body(i, _):
    ref[pl.ds(i * 16, 16)] = v
    return 0

  lax.fori_loop(0, n // 16, body, 0)


def _agg_body(h2_hbm, srci_hbm, dsti_hbm, lbl_hbm,
              ph_hbm, plb_hbm,
              idxb, hbuf, lbuf, sems,
              agg_h, agg_l):
  c = lax.axis_index("c")
  s = lax.axis_index("s")
  w = c * NS + s
  off = c * NTAB

  _fill2d(hbuf, ROW, HALF, 0.0, jnp.float32)
  _fill2d(lbuf, ROW, LBL_DIM, 0.0, jnp.float32)

  nb = s * NPT
  for k in range(NPT // ROW):
    sl = pl.ds(nb + k * ROW, ROW)
    pltpu.sync_copy(hbuf.at[pl.ds(0, ROW)], agg_h.at[sl])
    pltpu.sync_copy(lbuf.at[pl.ds(0, ROW)], agg_l.at[sl])
  plsc.subcore_barrier()

  def load_batch(r0, half):
    pltpu.sync_copy(srci_hbm.at[pl.ds(r0, G)], idxb.at[pl.ds(half * 2 * G, G)])
    pltpu.sync_copy(dsti_hbm.at[pl.ds(r0, G)],
                    idxb.at[pl.ds(half * 2 * G + G, G)])

    def fix(i, _):
      for jj in range(ROW // 16):
        sl = pl.ds(jj * 16, 16)
        row = half * 2 * G + i
        idxb[row, sl] = idxb[row, sl] + off
      return 0

    lax.fori_loop(0, G, fix, 0)

  def fire_gathers(half, sem):
    ds_ = []
    for k in range(G):
      ds_.append(pltpu.async_copy(
          h2_hbm.at[idxb.at[half * 2 * G + k]],
          hbuf.at[pl.ds((half * G + k) * ROW, ROW)], sem))
    return ds_

  def scatter_batch(half):
    for k in range(G):
      pltpu.sync_copy(hbuf.at[pl.ds((half * G + k) * ROW, ROW)],
                      agg_h.at[idxb.at[half * 2 * G + G + k]], add=True)

  def h_pair(q, _):
    r0 = s * RPT1 + q * 2 * G
    load_batch(r0, 0)
    da = fire_gathers(0, sems.at[0])
    load_batch(r0 + G, 1)
    db = fire_gathers(1, sems.at[1])
    for d in da:
      d.wait()
    scatter_batch(0)
    for d in db:
      d.wait()
    scatter_batch(1)
    return 0

  lax.fori_loop(0, RPT1 // (2 * G), h_pair, 0)

  def l_pair(q, _):
    r0 = w * RPT + q * 2 * LG
    ds_ = []
    for half in range(2):
      rh = r0 + half * LG
      pltpu.sync_copy(dsti_hbm.at[pl.ds(rh, LG)],
                      idxb.at[pl.ds(half * LG, LG)])
      ds_.append(pltpu.async_copy(
          lbl_hbm.at[pl.ds(rh * ROW, LG * ROW)],
          lbuf.at[pl.ds(half * LG * ROW, LG * ROW)], sems.at[half]))
    for half in range(2):
      ds_[half].wait()
      for k in range(LG):
        pltpu.sync_copy(lbuf.at[pl.ds((half * LG + k) * ROW, ROW)],
                        agg_l.at[idxb.at[half * LG + k]], add=True)
    return 0

  lax.fori_loop(0, RPT // (2 * LG), l_pair, 0)
  plsc.subcore_barrier()

  for k in range(NPT // ROW):
    sl = pl.ds(nb + k * ROW, ROW)
    pltpu.sync_copy(agg_h.at[sl], ph_hbm.at[c, sl])
    pltpu.sync_copy(agg_l.at[sl], plb_hbm.at[c, sl])


_agg = pl.kernel(
    _agg_body,
    out_type=[
        jax.ShapeDtypeStruct((NC, NACC, HALF), jnp.float32),
        jax.ShapeDtypeStruct((NC, NACC, LBL_DIM), jnp.float32),
    ],
    mesh=plsc.VectorSubcoreMesh(core_axis_name="c", subcore_axis_name="s"),
    compiler_params=pltpu.CompilerParams(use_tc_tiling_on_sc=False),
    scratch_types=[
        pltpu.VMEM((4 * G, ROW), jnp.int32),
        pltpu.VMEM((2 * G * ROW, HALF), jnp.float32),
        pltpu.VMEM((2 * LG * ROW, LBL_DIM), jnp.float32),
        pltpu.SemaphoreType.DMA((2,)),
        pltpu.VMEM_SHARED((NACC, HALF), jnp.float32),
        pltpu.VMEM_SHARED((NACC, LBL_DIM), jnp.float32),
    ],
)


def _denom_body(dsti_hbm, inv_hbm, bkt_hbm,
                dst_all, ones_v, zb, dvbuf, degi, dbuf, invbuf, sem,
                bucket_sp, deg_sp):
  s = lax.axis_index("s")

  _fill1d(zb, 2048, 0.0, jnp.float32)
  _fill1d(ones_v, ROW, 1.0, jnp.float32)
  for k in range(BPT // 2048):
    pltpu.sync_copy(zb, bucket_sp.at[pl.ds(s * BPT + k * 2048, 2048)])
  pltpu.sync_copy(zb.at[pl.ds(0, NPT)], deg_sp.at[pl.ds(s * NPT, NPT)])
  pltpu.sync_copy(dsti_hbm.at[pl.ds(s * RPT1, RPT1)], dst_all)
  plsc.subcore_barrier()

  def deg_batch(q, _):
    ds_ = []
    for k in range(8):
      ds_.append(pltpu.async_copy(
          ones_v, deg_sp.at[dst_all.at[q * 8 + k]], sem, add=True))
    for d in ds_:
      d.wait()
    return 0

  lax.fori_loop(0, RPT1 // 8, deg_batch, 0)
  plsc.subcore_barrier()

  n0 = s * NPT
  pltpu.sync_copy(deg_sp.at[pl.ds(n0, NPT)], dvbuf)
  for i in range(NPT // 16):
    x = dvbuf[pl.ds(i * 16, 16)]
    n = n0 + i * 16 + lax.iota(jnp.int32, 16)
    idx = jnp.where(n < N_NODES, x.astype(jnp.int32), SENT)
    degi[i // 8, pl.ds((i % 8) * 16, 16)] = idx

  ds_ = []
  for k in range(NPT // ROW):
    ds_.append(pltpu.async_copy(ones_v, bucket_sp.at[degi.at[k]], sem,
                                add=True))
  for d in ds_:
    d.wait()
  plsc.subcore_barrier()

  pltpu.sync_copy(bucket_sp.at[pl.ds(s * BPT, BPT)], bkt_hbm.at[pl.ds(s * BPT, BPT)])
  plsc.subcore_barrier()

  ds_ = []
  for k in range(NPT // ROW):
    ds_.append(pltpu.async_copy(bkt_hbm.at[degi.at[k]],
                                dbuf.at[pl.ds(k * ROW, ROW)], sem))
  for d in ds_:
    d.wait()
  for i in range(NPT // 16):
    d = dbuf[pl.ds(i * 16, 16)]
    invbuf[pl.ds(i * 16, 16)] = 1.0 / d
  pltpu.sync_copy(invbuf, inv_hbm.at[pl.ds(n0, NPT)])


_denom = pl.kernel(
    _denom_body,
    out_type=[
        jax.ShapeDtypeStruct((NACC,), jnp.float32),
        jax.ShapeDtypeStruct((NB,), jnp.float32),
    ],
    mesh=plsc.VectorSubcoreMesh(core_axis_name="c", subcore_axis_name="s"),
    scratch_types=[
        pltpu.VMEM((RPT1, ROW), jnp.int32),
        pltpu.VMEM((ROW,), jnp.float32),
        pltpu.VMEM((2048,), jnp.float32),
        pltpu.VMEM((NPT,), jnp.float32),
        pltpu.VMEM((NPT // ROW, ROW), jnp.int32),
        pltpu.VMEM((NPT,), jnp.float32),
        pltpu.VMEM((NPT,), jnp.float32),
        pltpu.SemaphoreType.DMA,
        pltpu.VMEM_SHARED((NB,), jnp.float32),
        pltpu.VMEM_SHARED((NACC,), jnp.float32),
    ],
)


def _linear_body(ph, plb, inv, w1, w2, o):
  b = plb[0] + plb[1]
  acc = lax.dot_general(ph[0], w1[:, :HALF], (((1,), (1,)), ((), ())),
                        preferred_element_type=jnp.float32,
                        precision=lax.Precision.HIGHEST)
  acc = acc + lax.dot_general(ph[1], w1[:, HALF:], (((1,), (1,)), ((), ())),
                              preferred_element_type=jnp.float32,
                              precision=lax.Precision.HIGHEST)
  acc = acc + lax.dot_general(b, w2[...], (((1,), (1,)), ((), ())),
                              preferred_element_type=jnp.float32,
                              precision=lax.Precision.HIGHEST)
  acc = acc * inv[...]
  o[...] = jnp.where(acc > 0, acc, jnp.exp(jnp.minimum(acc, 0.0)) - 1.0)


_TC_BLK = 1024


def _linear(ph, plb, inv, w1, w2):
  grid = (NACC // _TC_BLK,)
  return pl.pallas_call(
      _linear_body,
      grid=grid,
      in_specs=[
          pl.BlockSpec((NC, _TC_BLK, HALF), lambda i: (0, i, 0)),
          pl.BlockSpec((NC, _TC_BLK, LBL_DIM), lambda i: (0, i, 0)),
          pl.BlockSpec((_TC_BLK, 1), lambda i: (i, 0)),
          pl.BlockSpec((Z_DIM, NODE_DIM), lambda i: (0, 0)),
          pl.BlockSpec((Z_DIM, LBL_DIM), lambda i: (0, 0)),
      ],
      out_specs=pl.BlockSpec((_TC_BLK, Z_DIM), lambda i: (i, 0)),
      out_shape=jax.ShapeDtypeStruct((NACC, Z_DIM), jnp.float32),
  )(ph, plb, inv, w1, w2)


@jax.jit
def kernel(h, edge_index, edge_lbl, W):
  pad = EP - N_EDGES
  src = edge_index[0]
  dst = edge_index[1]
  fill = jnp.full((pad,), N_NODES, jnp.int32)
  srcp = jnp.concatenate([src, fill]).reshape(NROWS, ROW)
  dstp = jnp.concatenate([dst, fill]).reshape(NROWS, ROW)
  lblp = jnp.concatenate(
      [edge_lbl, jnp.zeros((pad, LBL_DIM), jnp.float32)], axis=0)
  hp = jnp.concatenate([h, jnp.zeros((1, NODE_DIM), jnp.float32)], axis=0)
  h2 = jnp.concatenate([hp[:, :HALF], hp[:, HALF:]], axis=0)

  ph, plb = _agg(h2, srcp, dstp, lblp)
  inv, _ = _denom(dstp)

  z = _linear(ph, plb, inv.reshape(NACC, 1), W[:, :NODE_DIM], W[:, NODE_DIM:])
  return z[:N_NODES]
